# baseline (device time: 16160 ns/iter reference)
import jax
import jax.numpy as jnp
from jax import lax
from jax.experimental import pallas as pl
from jax.experimental.pallas import tpu as pltpu

N_CHUNKS = 16


def kernel(x):
    m_per, n = x.shape
    half = m_per // 2
    rows = half // N_CHUNKS

    def body(
        x_ref,
        out_hbm,
        loc,
        ybuf,
        zbuf,
        mine_sem,
        yout_sems,
        zout_sems,
        ysend,
        yrecv,
        zsend,
        zrecv,
    ):
        my_x = lax.axis_index("x")
        my_y = lax.axis_index("y")
        my_z = lax.axis_index("z")
        ynbr = (my_x, 1 - my_y, my_z)
        znbr = (my_x, my_y, 1 - my_z)

        mine = my_y * m_per
        theirs = (1 - my_y) * m_per
        y_in = theirs + my_z * half
        z_in = theirs + (1 - my_z) * half

        barrier = pltpu.get_barrier_semaphore()
        for nbr in (ynbr, znbr):
            pl.semaphore_signal(
                barrier, inc=1, device_id=nbr,
                device_id_type=pl.DeviceIdType.MESH,
            )

        loc[pl.ds(my_z * half, half), :] = x_ref[
            pl.ds(my_z * half, half), :
        ].astype(jnp.bfloat16)

        pl.semaphore_wait(barrier, 2)

        y_rdmas = []
        for c in range(N_CHUNKS):
            rdma = pltpu.make_async_remote_copy(
                src_ref=loc.at[pl.ds(my_z * half + c * rows, rows)],
                dst_ref=ybuf.at[c],
                send_sem=ysend.at[c],
                recv_sem=yrecv.at[c],
                device_id=ynbr,
                device_id_type=pl.DeviceIdType.MESH,
            )
            rdma.start()
            y_rdmas.append(rdma)

        loc[pl.ds((1 - my_z) * half, half), :] = x_ref[
            pl.ds((1 - my_z) * half, half), :
        ].astype(jnp.bfloat16)
        cpmine = pltpu.make_async_copy(
            loc, out_hbm.at[pl.ds(mine, m_per)], mine_sem
        )
        cpmine.start()

        z_rdmas = []
        y_outs = []
        for c in range(N_CHUNKS):
            y_rdmas[c].wait_recv()
            rdma = pltpu.make_async_remote_copy(
                src_ref=ybuf.at[c],
                dst_ref=zbuf.at[c],
                send_sem=zsend.at[c],
                recv_sem=zrecv.at[c],
                device_id=znbr,
                device_id_type=pl.DeviceIdType.MESH,
            )
            rdma.start()
            z_rdmas.append(rdma)
            cp = pltpu.make_async_copy(
                ybuf.at[c],
                out_hbm.at[pl.ds(y_in + c * rows, rows)],
                yout_sems.at[c],
            )
            cp.start()
            y_outs.append(cp)

        z_outs = []
        for c in range(N_CHUNKS):
            z_rdmas[c].wait_recv()
            cp = pltpu.make_async_copy(
                zbuf.at[c],
                out_hbm.at[pl.ds(z_in + c * rows, rows)],
                zout_sems.at[c],
            )
            cp.start()
            z_outs.append(cp)

        cpmine.wait()
        for c in range(N_CHUNKS):
            y_outs[c].wait()
            z_outs[c].wait()
            y_rdmas[c].wait_send()
            z_rdmas[c].wait_send()

    return pl.pallas_call(
        body,
        out_shape=jax.ShapeDtypeStruct((2 * m_per, n), jnp.bfloat16),
        in_specs=[pl.BlockSpec(memory_space=pltpu.VMEM)],
        out_specs=pl.BlockSpec(memory_space=pl.ANY),
        scratch_shapes=[
            pltpu.VMEM((m_per, n), jnp.bfloat16),
            pltpu.VMEM((N_CHUNKS, rows, n), jnp.bfloat16),
            pltpu.VMEM((N_CHUNKS, rows, n), jnp.bfloat16),
            pltpu.SemaphoreType.DMA,
            pltpu.SemaphoreType.DMA((N_CHUNKS,)),
            pltpu.SemaphoreType.DMA((N_CHUNKS,)),
            pltpu.SemaphoreType.DMA((N_CHUNKS,)),
            pltpu.SemaphoreType.DMA((N_CHUNKS,)),
            pltpu.SemaphoreType.DMA((N_CHUNKS,)),
            pltpu.SemaphoreType.DMA((N_CHUNKS,)),
        ],
        compiler_params=pltpu.CompilerParams(collective_id=0),
    )(x)


# device time: 15781 ns/iter; 1.0240x vs baseline; 1.0240x over previous
import jax
import jax.numpy as jnp
from jax import lax
from jax.experimental import pallas as pl
from jax.experimental.pallas import tpu as pltpu

N_CHUNKS = 8


def kernel(x):
    m_per, n = x.shape
    half = m_per // 2
    rows = half // N_CHUNKS

    def body(x_ref, out_ref, zbar, ysend, yrecv, zsend, zrecv):
        my_x = lax.axis_index("x")
        my_y = lax.axis_index("y")
        my_z = lax.axis_index("z")
        ynbr = (my_x, 1 - my_y, my_z)
        znbr = (my_x, my_y, 1 - my_z)

        mine = my_y * m_per
        y_half = mine + my_z * half
        theirs = (1 - my_y) * m_per
        y_in = theirs + my_z * half

        barrier = pltpu.get_barrier_semaphore()
        pl.semaphore_signal(
            barrier, inc=1, device_id=ynbr,
            device_id_type=pl.DeviceIdType.MESH,
        )
        pl.semaphore_signal(
            zbar, inc=1, device_id=znbr,
            device_id_type=pl.DeviceIdType.MESH,
        )

        out_ref[pl.ds(y_half, half), :] = x_ref[
            pl.ds(my_z * half, half), :
        ].astype(jnp.bfloat16)
        pl.semaphore_wait(barrier, 1)

        y_rdmas = []
        for c in range(N_CHUNKS):
            rdma = pltpu.make_async_remote_copy(
                src_ref=out_ref.at[pl.ds(y_half + c * rows, rows)],
                dst_ref=out_ref.at[pl.ds(y_half + c * rows, rows)],
                send_sem=ysend.at[c],
                recv_sem=yrecv.at[c],
                device_id=ynbr,
                device_id_type=pl.DeviceIdType.MESH,
            )
            rdma.start()
            y_rdmas.append(rdma)

        out_ref[pl.ds(mine + (1 - my_z) * half, half), :] = x_ref[
            pl.ds((1 - my_z) * half, half), :
        ].astype(jnp.bfloat16)

        pl.semaphore_wait(zbar, 1)

        z_rdmas = []
        for c in range(N_CHUNKS):
            y_rdmas[c].wait_recv()
            rdma = pltpu.make_async_remote_copy(
                src_ref=out_ref.at[pl.ds(y_in + c * rows, rows)],
                dst_ref=out_ref.at[pl.ds(y_in + c * rows, rows)],
                send_sem=zsend.at[c],
                recv_sem=zrecv.at[c],
                device_id=znbr,
                device_id_type=pl.DeviceIdType.MESH,
            )
            rdma.start()
            z_rdmas.append(rdma)

        for c in range(N_CHUNKS):
            z_rdmas[c].wait_recv()
        for c in range(N_CHUNKS):
            y_rdmas[c].wait_send()
            z_rdmas[c].wait_send()

    return pl.pallas_call(
        body,
        out_shape=jax.ShapeDtypeStruct((2 * m_per, n), jnp.bfloat16),
        in_specs=[pl.BlockSpec(memory_space=pltpu.VMEM)],
        out_specs=pl.BlockSpec(memory_space=pltpu.VMEM),
        scratch_shapes=[
            pltpu.SemaphoreType.REGULAR,
            pltpu.SemaphoreType.DMA((N_CHUNKS,)),
            pltpu.SemaphoreType.DMA((N_CHUNKS,)),
            pltpu.SemaphoreType.DMA((N_CHUNKS,)),
            pltpu.SemaphoreType.DMA((N_CHUNKS,)),
        ],
        compiler_params=pltpu.CompilerParams(collective_id=0),
    )(x)
